# Initial kernel scaffold; baseline (speedup 1.0000x reference)
import functools

import jax
import jax.numpy as jnp
from jax import lax
from jax.experimental import pallas as pl
from jax.experimental.pallas import tpu as pltpu
from jax.experimental.pallas import tpu_sc as plsc

NC = 2
NS = 16
LANES = 16
NW = NC * NS


def _vector_mesh():
    return plsc.VectorSubcoreMesh(core_axis_name="c", subcore_axis_name="s")


def _sc_counts(dst3, n):
    _, C, K = dst3.shape
    rpt = n // NS

    @functools.partial(
        pl.kernel,
        out_type=jax.ShapeDtypeStruct((NC, n, LANES), jnp.float32),
        mesh=_vector_mesh(),
        scratch_types=[
            pltpu.VMEM((K,), jnp.int32),
            pltpu.VMEM((K, LANES), jnp.float32),
            pltpu.VMEM((rpt, LANES), jnp.float32),
            pltpu.VMEM_SHARED((n, LANES), jnp.float32),
        ],
    )
    def k(dst_hbm, out_hbm, idx_v, ones_v, zero_v, acc_s):
        cid = lax.axis_index("c")
        sid = lax.axis_index("s")
        wid = sid * NC + cid

        @pl.loop(0, K)
        def _(i):
            ones_v[i] = jnp.ones((LANES,), jnp.float32)

        @pl.loop(0, rpt)
        def _(i):
            zero_v[i] = jnp.zeros((LANES,), jnp.float32)

        base = sid * rpt
        pltpu.sync_copy(zero_v, acc_s.at[pl.ds(base, rpt)])
        plsc.subcore_barrier()

        @pl.loop(0, C)
        def _(j):
            pltpu.sync_copy(dst_hbm.at[wid, j], idx_v)
            pltpu.sync_copy(ones_v, acc_s.at[idx_v], add=True)

        plsc.subcore_barrier()
        pltpu.sync_copy(acc_s.at[pl.ds(base, rpt)],
                        out_hbm.at[cid, pl.ds(base, rpt)])

    return k(dst3)


def _sc_agg(table, src3, dst3):
    n, h = table.shape
    _, C, K = src3.shape
    rpt = n // NS

    @functools.partial(
        pl.kernel,
        out_type=jax.ShapeDtypeStruct((NC, n, h), jnp.float32),
        mesh=_vector_mesh(),
        scratch_types=[
            pltpu.VMEM((K,), jnp.int32),
            pltpu.VMEM((K,), jnp.int32),
            pltpu.VMEM((K, h), jnp.float32),
            pltpu.VMEM((rpt, h), jnp.float32),
            pltpu.VMEM_SHARED((n, h), jnp.float32),
            pltpu.SemaphoreType.DMA,
        ],
    )
    def k(table_hbm, src_hbm, dst_hbm, out_hbm, s_v, d_v, rows_v, zero_v,
          acc_s, sem):
        cid = lax.axis_index("c")
        sid = lax.axis_index("s")
        wid = sid * NC + cid

        @pl.loop(0, rpt)
        def _(i):
            @pl.loop(0, h, step=LANES)
            def _(c):
                zero_v[i, pl.ds(c, LANES)] = jnp.zeros((LANES,), jnp.float32)

        base = sid * rpt
        pltpu.sync_copy(zero_v, acc_s.at[pl.ds(base, rpt)])
        plsc.subcore_barrier()

        @pl.loop(0, C)
        def _(j):
            pltpu.sync_copy(src_hbm.at[wid, j], s_v)
            pltpu.async_copy(table_hbm.at[s_v], rows_v, sem).wait()
            pltpu.sync_copy(dst_hbm.at[wid, j], d_v)
            pltpu.sync_copy(rows_v, acc_s.at[d_v], add=True)

        plsc.subcore_barrier()
        pltpu.sync_copy(acc_s.at[pl.ds(base, rpt)],
                        out_hbm.at[cid, pl.ds(base, rpt)])

    return k(table, src3, dst3)


def _sc_gather2(a_tab, b_tab, src3, dst3):
    n, h = a_tab.shape
    _, C, K = src3.shape
    ew = C * K
    e = NW * ew
    out_sd = jax.ShapeDtypeStruct((e, h), jnp.float32)

    @functools.partial(
        pl.kernel,
        out_type=(out_sd, out_sd),
        mesh=_vector_mesh(),
        scratch_types=[
            pltpu.VMEM((K,), jnp.int32),
            pltpu.VMEM((K,), jnp.int32),
            pltpu.VMEM((K, h), jnp.float32),
            pltpu.VMEM((K, h), jnp.float32),
            pltpu.SemaphoreType.DMA,
            pltpu.SemaphoreType.DMA,
        ],
    )
    def k(a_hbm, b_hbm, src_hbm, dst_hbm, oa_hbm, ob_hbm, s_v, d_v,
          ra_v, rb_v, sem_a, sem_b):
        cid = lax.axis_index("c")
        sid = lax.axis_index("s")
        wid = sid * NC + cid

        @pl.loop(0, C)
        def _(j):
            base = wid * ew + j * K
            pltpu.sync_copy(src_hbm.at[wid, j], s_v)
            cp_a = pltpu.async_copy(a_hbm.at[s_v], ra_v, sem_a)
            pltpu.sync_copy(dst_hbm.at[wid, j], d_v)
            cp_b = pltpu.async_copy(b_hbm.at[d_v], rb_v, sem_b)
            cp_a.wait()
            pltpu.sync_copy(ra_v, oa_hbm.at[pl.ds(base, K)])
            cp_b.wait()
            pltpu.sync_copy(rb_v, ob_hbm.at[pl.ds(base, K)])

    return k(a_tab, b_tab, src3, dst3)


def _tc_mm(xx, ww, bn=1000):
    n, d = xx.shape
    h = ww.shape[1]

    def body(x_ref, w_ref, o_ref):
        o_ref[...] = jnp.dot(x_ref[...], w_ref[...],
                             preferred_element_type=jnp.float32)

    return pl.pallas_call(
        body,
        grid=(n // bn,),
        in_specs=[pl.BlockSpec((bn, d), lambda i: (i, 0)),
                  pl.BlockSpec((d, h), lambda i: (0, 0))],
        out_specs=pl.BlockSpec((bn, h), lambda i: (i, 0)),
        out_shape=jax.ShapeDtypeStruct((n, h), jnp.float32),
    )(xx, ww)


def _tc_scale(h1, cnt, bn=1000):
    n, h = h1.shape

    def body(h_ref, c_ref, o_ref, dinv_ref):
        deg = 1.0 + c_ref[0, :, 0:1] + c_ref[1, :, 0:1]
        dinv = lax.rsqrt(deg)
        o_ref[...] = h_ref[...] * dinv
        dinv_ref[...] = dinv

    return pl.pallas_call(
        body,
        grid=(n // bn,),
        in_specs=[pl.BlockSpec((bn, h), lambda i: (i, 0)),
                  pl.BlockSpec((NC, bn, LANES), lambda i: (0, i, 0))],
        out_specs=[pl.BlockSpec((bn, h), lambda i: (i, 0)),
                   pl.BlockSpec((bn, 1), lambda i: (i, 0))],
        out_shape=[jax.ShapeDtypeStruct((n, h), jnp.float32),
                   jax.ShapeDtypeStruct((n, 1), jnp.float32)],
    )(h1, cnt)


def _tc_layer(agg, hs, dinv, bias, ww, relu, bn=1000):
    n, h = hs.shape

    def body(a_ref, hs_ref, dinv_ref, b_ref, w_ref, o_ref):
        z = dinv_ref[...] * (a_ref[0] + a_ref[1] + hs_ref[...]) + b_ref[...]
        if relu:
            z = jnp.maximum(z, 0.0)
        o_ref[...] = jnp.dot(z, w_ref[...],
                             preferred_element_type=jnp.float32) * dinv_ref[...]

    return pl.pallas_call(
        body,
        grid=(n // bn,),
        in_specs=[pl.BlockSpec((NC, bn, h), lambda i: (0, i, 0)),
                  pl.BlockSpec((bn, h), lambda i: (i, 0)),
                  pl.BlockSpec((bn, 1), lambda i: (i, 0)),
                  pl.BlockSpec((1, h), lambda i: (0, 0)),
                  pl.BlockSpec((h, h), lambda i: (0, 0))],
        out_specs=pl.BlockSpec((bn, h), lambda i: (i, 0)),
        out_shape=jax.ShapeDtypeStruct((n, h), jnp.float32),
    )(agg, hs, dinv, bias, ww)


def _tc_decode_tables(agg, hs, dinv, bias, wa, wb, l1b, bn=1000):
    n, h = hs.shape

    def body(a_ref, hs_ref, dinv_ref, b_ref, wa_ref, wb_ref, l1b_ref,
             oa_ref, ob_ref):
        z = dinv_ref[...] * (a_ref[0] + a_ref[1] + hs_ref[...]) + b_ref[...]
        oa_ref[...] = jnp.dot(z, wa_ref[...],
                              preferred_element_type=jnp.float32) + l1b_ref[...]
        ob_ref[...] = jnp.dot(z, wb_ref[...],
                              preferred_element_type=jnp.float32)

    return pl.pallas_call(
        body,
        grid=(n // bn,),
        in_specs=[pl.BlockSpec((NC, bn, h), lambda i: (0, i, 0)),
                  pl.BlockSpec((bn, h), lambda i: (i, 0)),
                  pl.BlockSpec((bn, 1), lambda i: (i, 0)),
                  pl.BlockSpec((1, h), lambda i: (0, 0)),
                  pl.BlockSpec((h, h), lambda i: (0, 0)),
                  pl.BlockSpec((h, h), lambda i: (0, 0)),
                  pl.BlockSpec((1, h), lambda i: (0, 0))],
        out_specs=[pl.BlockSpec((bn, h), lambda i: (i, 0)),
                   pl.BlockSpec((bn, h), lambda i: (i, 0))],
        out_shape=[jax.ShapeDtypeStruct((n, h), jnp.float32),
                   jax.ShapeDtypeStruct((n, h), jnp.float32)],
    )(agg, hs, dinv, bias, wa, wb, l1b)


def _tc_decode(asrc, bdst, w2, b2, bn=4000):
    e, h = asrc.shape

    def body(a_ref, b_ref, w_ref, bb_ref, o_ref):
        z = jnp.maximum(a_ref[...] + b_ref[...], 0.0)
        o_ref[...] = jax.nn.sigmoid(
            jnp.dot(z, w_ref[...], preferred_element_type=jnp.float32)
            + bb_ref[...])

    return pl.pallas_call(
        body,
        grid=(e // bn,),
        in_specs=[pl.BlockSpec((bn, h), lambda i: (i, 0)),
                  pl.BlockSpec((bn, h), lambda i: (i, 0)),
                  pl.BlockSpec((h, 1), lambda i: (0, 0)),
                  pl.BlockSpec((1, 1), lambda i: (0, 0))],
        out_specs=pl.BlockSpec((bn, 1), lambda i: (i, 0)),
        out_shape=jax.ShapeDtypeStruct((e, 1), jnp.float32),
    )(asrc, bdst, w2, b2)


def kernel(x, edge_index, W1, b1, W2, b2, lin1_W, lin1_b, lin2_W, lin2_b):
    n, _ = x.shape
    h = W1.shape[1]
    e = edge_index.shape[1]

    ew = e // NW
    K = 80
    C = ew // K

    src3 = edge_index[0].reshape(NW, C, K)
    dst3 = edge_index[1].reshape(NW, C, K)

    cnt = _sc_counts(dst3, n)
    h1 = _tc_mm(x, W1)
    h1s, dinv = _tc_scale(h1, cnt)

    agg1 = _sc_agg(h1s, src3, dst3)
    h2s = _tc_layer(agg1, h1s, dinv, b1.reshape(1, h), W2, relu=True)

    agg2 = _sc_agg(h2s, src3, dst3)
    a_tab, b_tab = _tc_decode_tables(agg2, h2s, dinv, b2.reshape(1, h),
                                     lin1_W[:h], lin1_W[h:],
                                     lin1_b.reshape(1, h))

    asrc, bdst = _sc_gather2(a_tab, b_tab, src3, dst3)
    return _tc_decode(asrc, bdst, lin2_W, lin2_b.reshape(1, 1))


# speedup vs baseline: 7.7461x; 7.7461x over previous
import functools

import jax
import jax.numpy as jnp
from jax import lax
from jax.experimental import pallas as pl
from jax.experimental.pallas import tpu as pltpu
from jax.experimental.pallas import tpu_sc as plsc

NC = 2
NS = 16
LANES = 16
NW = NC * NS


def _vector_mesh():
    return plsc.VectorSubcoreMesh(core_axis_name="c", subcore_axis_name="s")


_SC_PARAMS = pltpu.CompilerParams(use_tc_tiling_on_sc=False)


def _sc_counts(dst3, n):
    _, C, K = dst3.shape
    np_ = ((n + NS * 8 - 1) // (NS * 8)) * (NS * 8)
    rpt = np_ // NS

    @functools.partial(
        pl.kernel,
        out_type=jax.ShapeDtypeStruct((NC, np_, LANES), jnp.float32),
        mesh=_vector_mesh(),
        compiler_params=_SC_PARAMS,
        scratch_types=[
            pltpu.VMEM((K,), jnp.int32),
            pltpu.VMEM((K, LANES), jnp.float32),
            pltpu.VMEM((rpt, LANES), jnp.float32),
            pltpu.VMEM_SHARED((np_, LANES), jnp.float32),
        ],
    )
    def k(dst_hbm, out_hbm, idx_v, ones_v, zero_v, acc_s):
        cid = lax.axis_index("c")
        sid = lax.axis_index("s")
        wid = sid * NC + cid

        @pl.loop(0, K)
        def _(i):
            ones_v[i] = jnp.ones((LANES,), jnp.float32)

        @pl.loop(0, rpt)
        def _(i):
            zero_v[i] = jnp.zeros((LANES,), jnp.float32)

        base = sid * rpt
        pltpu.sync_copy(zero_v, acc_s.at[pl.ds(base, rpt)])
        plsc.subcore_barrier()

        @pl.loop(0, C)
        def _(j):
            pltpu.sync_copy(dst_hbm.at[wid, j], idx_v)
            pltpu.sync_copy(ones_v, acc_s.at[idx_v], add=True)

        plsc.subcore_barrier()
        pltpu.sync_copy(acc_s.at[pl.ds(base, rpt)],
                        out_hbm.at[cid, pl.ds(base, rpt)])

    return k(dst3)


def _sc_agg(table, src3, dst3):
    n, h = table.shape
    _, C, K = src3.shape
    np_ = ((n + NS * 8 - 1) // (NS * 8)) * (NS * 8)
    rpt = np_ // NS

    @functools.partial(
        pl.kernel,
        out_type=jax.ShapeDtypeStruct((NC, np_, h), jnp.float32),
        mesh=_vector_mesh(),
        compiler_params=_SC_PARAMS,
        scratch_types=[
            pltpu.VMEM((K,), jnp.int32),
            pltpu.VMEM((K,), jnp.int32),
            pltpu.VMEM((K, h), jnp.float32),
            pltpu.VMEM((rpt, h), jnp.float32),
            pltpu.VMEM_SHARED((np_, h), jnp.float32),
            pltpu.SemaphoreType.DMA,
        ],
    )
    def k(table_hbm, src_hbm, dst_hbm, out_hbm, s_v, d_v, rows_v, zero_v,
          acc_s, sem):
        cid = lax.axis_index("c")
        sid = lax.axis_index("s")
        wid = sid * NC + cid

        @pl.loop(0, rpt)
        def _(i):
            @pl.loop(0, h, step=LANES)
            def _(c):
                zero_v[i, pl.ds(c, LANES)] = jnp.zeros((LANES,), jnp.float32)

        base = sid * rpt
        pltpu.sync_copy(zero_v, acc_s.at[pl.ds(base, rpt)])
        plsc.subcore_barrier()

        @pl.loop(0, C)
        def _(j):
            pltpu.sync_copy(src_hbm.at[wid, j], s_v)
            pltpu.async_copy(table_hbm.at[s_v], rows_v, sem).wait()
            pltpu.sync_copy(dst_hbm.at[wid, j], d_v)
            pltpu.sync_copy(rows_v, acc_s.at[d_v], add=True)

        plsc.subcore_barrier()
        pltpu.sync_copy(acc_s.at[pl.ds(base, rpt)],
                        out_hbm.at[cid, pl.ds(base, rpt)])

    return k(table, src3, dst3)


def _sc_gather2(a_tab, b_tab, src3, dst3):
    n, h = a_tab.shape
    _, C, K = src3.shape
    ew = C * K
    e = NW * ew
    out_sd = jax.ShapeDtypeStruct((e, h), jnp.float32)

    @functools.partial(
        pl.kernel,
        out_type=(out_sd, out_sd),
        mesh=_vector_mesh(),
        compiler_params=_SC_PARAMS,
        scratch_types=[
            pltpu.VMEM((K,), jnp.int32),
            pltpu.VMEM((K,), jnp.int32),
            pltpu.VMEM((K, h), jnp.float32),
            pltpu.VMEM((K, h), jnp.float32),
            pltpu.SemaphoreType.DMA,
            pltpu.SemaphoreType.DMA,
        ],
    )
    def k(a_hbm, b_hbm, src_hbm, dst_hbm, oa_hbm, ob_hbm, s_v, d_v,
          ra_v, rb_v, sem_a, sem_b):
        cid = lax.axis_index("c")
        sid = lax.axis_index("s")
        wid = sid * NC + cid

        @pl.loop(0, C)
        def _(j):
            base = wid * ew + j * K
            pltpu.sync_copy(src_hbm.at[wid, j], s_v)
            cp_a = pltpu.async_copy(a_hbm.at[s_v], ra_v, sem_a)
            pltpu.sync_copy(dst_hbm.at[wid, j], d_v)
            cp_b = pltpu.async_copy(b_hbm.at[d_v], rb_v, sem_b)
            cp_a.wait()
            pltpu.sync_copy(ra_v, oa_hbm.at[pl.ds(base, K)])
            cp_b.wait()
            pltpu.sync_copy(rb_v, ob_hbm.at[pl.ds(base, K)])

    return k(a_tab, b_tab, src3, dst3)


def _tc_mm(xx, ww, bn=1000):
    n, d = xx.shape
    h = ww.shape[1]

    def body(x_ref, w_ref, o_ref):
        o_ref[...] = jnp.dot(x_ref[...], w_ref[...],
                             preferred_element_type=jnp.float32)

    return pl.pallas_call(
        body,
        grid=(n // bn,),
        in_specs=[pl.BlockSpec((bn, d), lambda i: (i, 0)),
                  pl.BlockSpec((d, h), lambda i: (0, 0))],
        out_specs=pl.BlockSpec((bn, h), lambda i: (i, 0)),
        out_shape=jax.ShapeDtypeStruct((n, h), jnp.float32),
    )(xx, ww)


def _tc_scale(h1, cnt, bn=1000):
    n, h = h1.shape

    def body(h_ref, c_ref, o_ref, dinv_ref):
        deg = 1.0 + c_ref[0, :, 0:1] + c_ref[1, :, 0:1]
        dinv = lax.rsqrt(deg)
        o_ref[...] = h_ref[...] * dinv
        dinv_ref[...] = dinv

    return pl.pallas_call(
        body,
        grid=(n // bn,),
        in_specs=[pl.BlockSpec((bn, h), lambda i: (i, 0)),
                  pl.BlockSpec((NC, bn, LANES), lambda i: (0, i, 0))],
        out_specs=[pl.BlockSpec((bn, h), lambda i: (i, 0)),
                   pl.BlockSpec((bn, 1), lambda i: (i, 0))],
        out_shape=[jax.ShapeDtypeStruct((n, h), jnp.float32),
                   jax.ShapeDtypeStruct((n, 1), jnp.float32)],
    )(h1, cnt)


def _tc_layer(agg, hs, dinv, bias, ww, relu, bn=1000):
    n, h = hs.shape

    def body(a_ref, hs_ref, dinv_ref, b_ref, w_ref, o_ref):
        z = dinv_ref[...] * (a_ref[0] + a_ref[1] + hs_ref[...]) + b_ref[...]
        if relu:
            z = jnp.maximum(z, 0.0)
        o_ref[...] = jnp.dot(z, w_ref[...],
                             preferred_element_type=jnp.float32) * dinv_ref[...]

    return pl.pallas_call(
        body,
        grid=(n // bn,),
        in_specs=[pl.BlockSpec((NC, bn, h), lambda i: (0, i, 0)),
                  pl.BlockSpec((bn, h), lambda i: (i, 0)),
                  pl.BlockSpec((bn, 1), lambda i: (i, 0)),
                  pl.BlockSpec((1, h), lambda i: (0, 0)),
                  pl.BlockSpec((h, h), lambda i: (0, 0))],
        out_specs=pl.BlockSpec((bn, h), lambda i: (i, 0)),
        out_shape=jax.ShapeDtypeStruct((n, h), jnp.float32),
    )(agg, hs, dinv, bias, ww)


def _tc_decode_tables(agg, hs, dinv, bias, wa, wb, l1b, bn=1000):
    n, h = hs.shape

    def body(a_ref, hs_ref, dinv_ref, b_ref, wa_ref, wb_ref, l1b_ref,
             oa_ref, ob_ref):
        z = dinv_ref[...] * (a_ref[0] + a_ref[1] + hs_ref[...]) + b_ref[...]
        oa_ref[...] = jnp.dot(z, wa_ref[...],
                              preferred_element_type=jnp.float32) + l1b_ref[...]
        ob_ref[...] = jnp.dot(z, wb_ref[...],
                              preferred_element_type=jnp.float32)

    return pl.pallas_call(
        body,
        grid=(n // bn,),
        in_specs=[pl.BlockSpec((NC, bn, h), lambda i: (0, i, 0)),
                  pl.BlockSpec((bn, h), lambda i: (i, 0)),
                  pl.BlockSpec((bn, 1), lambda i: (i, 0)),
                  pl.BlockSpec((1, h), lambda i: (0, 0)),
                  pl.BlockSpec((h, h), lambda i: (0, 0)),
                  pl.BlockSpec((h, h), lambda i: (0, 0)),
                  pl.BlockSpec((1, h), lambda i: (0, 0))],
        out_specs=[pl.BlockSpec((bn, h), lambda i: (i, 0)),
                   pl.BlockSpec((bn, h), lambda i: (i, 0))],
        out_shape=[jax.ShapeDtypeStruct((n, h), jnp.float32),
                   jax.ShapeDtypeStruct((n, h), jnp.float32)],
    )(agg, hs, dinv, bias, wa, wb, l1b)


def _tc_decode(asrc, bdst, w2, b2, bn=4000):
    e, h = asrc.shape

    def body(a_ref, b_ref, w_ref, bb_ref, o_ref):
        z = jnp.maximum(a_ref[...] + b_ref[...], 0.0)
        o_ref[...] = jax.nn.sigmoid(
            jnp.dot(z, w_ref[...], preferred_element_type=jnp.float32)
            + bb_ref[...])

    return pl.pallas_call(
        body,
        grid=(e // bn,),
        in_specs=[pl.BlockSpec((bn, h), lambda i: (i, 0)),
                  pl.BlockSpec((bn, h), lambda i: (i, 0)),
                  pl.BlockSpec((h, 1), lambda i: (0, 0)),
                  pl.BlockSpec((1, 1), lambda i: (0, 0))],
        out_specs=pl.BlockSpec((bn, 1), lambda i: (i, 0)),
        out_shape=jax.ShapeDtypeStruct((e, 1), jnp.float32),
    )(asrc, bdst, w2, b2)


def kernel(x, edge_index, W1, b1, W2, b2, lin1_W, lin1_b, lin2_W, lin2_b):
    n, _ = x.shape
    h = W1.shape[1]
    e = edge_index.shape[1]

    ew = e // NW
    K = 80
    C = ew // K

    src3 = edge_index[0].reshape(NW, C, K)
    dst3 = edge_index[1].reshape(NW, C, K)

    cnt = _sc_counts(dst3, n)[:, :n]
    h1 = _tc_mm(x, W1)
    h1s, dinv = _tc_scale(h1, cnt)

    agg1 = _sc_agg(h1s, src3, dst3)[:, :n]
    h2s = _tc_layer(agg1, h1s, dinv, b1.reshape(1, h), W2, relu=True)

    agg2 = _sc_agg(h2s, src3, dst3)[:, :n]
    a_tab, b_tab = _tc_decode_tables(agg2, h2s, dinv, b2.reshape(1, h),
                                     lin1_W[:h], lin1_W[h:],
                                     lin1_b.reshape(1, h))

    asrc, bdst = _sc_gather2(a_tab, b_tab, src3, dst3)
    return _tc_decode(asrc, bdst, lin2_W, lin2_b.reshape(1, 1))


# speedup vs baseline: 12.4016x; 1.6010x over previous
import functools

import jax
import jax.numpy as jnp
from jax import lax
from jax.experimental import pallas as pl
from jax.experimental.pallas import tpu as pltpu
from jax.experimental.pallas import tpu_sc as plsc

NC = 2
NS = 16
LANES = 16
NW = NC * NS


def _vector_mesh():
    return plsc.VectorSubcoreMesh(core_axis_name="c", subcore_axis_name="s")


_SC_PARAMS = pltpu.CompilerParams(use_tc_tiling_on_sc=False)


def _padded_rows(n):
    return ((n + NS * 8 - 1) // (NS * 8)) * (NS * 8)


def _sc_counts(eidx, n):
    _, _, C, K = eidx.shape
    np_ = _padded_rows(n)
    rpt = np_ // NS

    @functools.partial(
        pl.kernel,
        out_type=jax.ShapeDtypeStruct((NC, np_, LANES), jnp.float32),
        mesh=_vector_mesh(),
        compiler_params=_SC_PARAMS,
        scratch_types=[
            pltpu.VMEM((C, K), jnp.int32),
            pltpu.VMEM((K, LANES), jnp.float32),
            pltpu.VMEM((rpt, LANES), jnp.float32),
            pltpu.VMEM_SHARED((np_, LANES), jnp.float32),
        ],
    )
    def k(eidx_hbm, out_hbm, idx_v, ones_v, zero_v, acc_s):
        cid = lax.axis_index("c")
        sid = lax.axis_index("s")
        wid = sid * NC + cid

        pltpu.sync_copy(eidx_hbm.at[wid, 1], idx_v)

        @pl.loop(0, K)
        def _(i):
            ones_v[i] = jnp.ones((LANES,), jnp.float32)

        @pl.loop(0, rpt)
        def _(i):
            zero_v[i] = jnp.zeros((LANES,), jnp.float32)

        base = sid * rpt
        pltpu.sync_copy(zero_v, acc_s.at[pl.ds(base, rpt)])
        plsc.subcore_barrier()

        @pl.loop(0, C)
        def _(j):
            pltpu.sync_copy(ones_v, acc_s.at[idx_v.at[j]], add=True)

        plsc.subcore_barrier()
        pltpu.sync_copy(acc_s.at[pl.ds(base, rpt)],
                        out_hbm.at[cid, pl.ds(base, rpt)])

    return k(eidx)


def _sc_agg(table, eidx):
    n, h = table.shape
    _, _, C, K = eidx.shape
    np_ = _padded_rows(n)
    rpt = np_ // NS

    @functools.partial(
        pl.kernel,
        out_type=jax.ShapeDtypeStruct((NC, np_, h), jnp.float32),
        mesh=_vector_mesh(),
        compiler_params=_SC_PARAMS,
        scratch_types=[
            pltpu.VMEM((2, C, K), jnp.int32),
            pltpu.VMEM((K, h), jnp.float32),
            pltpu.VMEM((K, h), jnp.float32),
            pltpu.VMEM((rpt, h), jnp.float32),
            pltpu.VMEM_SHARED((np_, h), jnp.float32),
            pltpu.SemaphoreType.DMA,
            pltpu.SemaphoreType.DMA,
        ],
    )
    def k(table_hbm, eidx_hbm, out_hbm, idx_v, rows_a, rows_b, zero_v,
          acc_s, sem_a, sem_b):
        cid = lax.axis_index("c")
        sid = lax.axis_index("s")
        wid = sid * NC + cid

        cp_idx = pltpu.async_copy(eidx_hbm.at[wid], idx_v, sem_a)

        @pl.loop(0, rpt)
        def _(i):
            @pl.loop(0, h, step=LANES)
            def _(c):
                zero_v[i, pl.ds(c, LANES)] = jnp.zeros((LANES,), jnp.float32)

        base = sid * rpt
        cp_idx.wait()
        pltpu.sync_copy(zero_v, acc_s.at[pl.ds(base, rpt)])
        plsc.subcore_barrier()

        pltpu.async_copy(table_hbm.at[idx_v.at[0, 0]], rows_a, sem_a)
        pltpu.async_copy(table_hbm.at[idx_v.at[0, 1]], rows_b, sem_b)

        @pl.loop(0, (C - 1) // 2)
        def _(m):
            j = 2 * m
            pltpu.make_async_copy(table_hbm.at[idx_v.at[0, 0]],
                                  rows_a, sem_a).wait()
            pltpu.sync_copy(rows_a, acc_s.at[idx_v.at[1, j]], add=True)

            @pl.when(j + 2 < C)
            def _():
                pltpu.async_copy(table_hbm.at[idx_v.at[0, j + 2]],
                                 rows_a, sem_a)

            pltpu.make_async_copy(table_hbm.at[idx_v.at[0, 1]],
                                  rows_b, sem_b).wait()
            pltpu.sync_copy(rows_b, acc_s.at[idx_v.at[1, j + 1]], add=True)

            @pl.when(j + 3 < C)
            def _():
                pltpu.async_copy(table_hbm.at[idx_v.at[0, j + 3]],
                                 rows_b, sem_b)

        if C % 2 == 1:
            pltpu.make_async_copy(table_hbm.at[idx_v.at[0, 0]],
                                  rows_a, sem_a).wait()
            pltpu.sync_copy(rows_a, acc_s.at[idx_v.at[1, C - 1]], add=True)

        plsc.subcore_barrier()
        pltpu.sync_copy(acc_s.at[pl.ds(base, rpt)],
                        out_hbm.at[cid, pl.ds(base, rpt)])

    return k(table, eidx)


def _sc_gather2(a_tab, b_tab, eidx):
    n, h = a_tab.shape
    _, _, C, K = eidx.shape
    ew = C * K
    e = NW * ew
    out_sd = jax.ShapeDtypeStruct((e, h), jnp.float32)

    @functools.partial(
        pl.kernel,
        out_type=(out_sd, out_sd),
        mesh=_vector_mesh(),
        compiler_params=_SC_PARAMS,
        scratch_types=[
            pltpu.VMEM((2, C, K), jnp.int32),
            pltpu.VMEM((2, K, h), jnp.float32),
            pltpu.VMEM((2, K, h), jnp.float32),
            [pltpu.SemaphoreType.DMA] * 4,
            [pltpu.SemaphoreType.DMA] * 4,
        ],
    )
    def k(a_hbm, b_hbm, eidx_hbm, oa_hbm, ob_hbm, idx_v, bufs_a, bufs_b,
          gsems, wsems):
        cid = lax.axis_index("c")
        sid = lax.axis_index("s")
        wid = sid * NC + cid

        pltpu.sync_copy(eidx_hbm.at[wid], idx_v)

        def gather(j, slot):
            pltpu.async_copy(a_hbm.at[idx_v.at[0, j]], bufs_a.at[slot],
                             gsems[slot])
            pltpu.async_copy(b_hbm.at[idx_v.at[1, j]], bufs_b.at[slot],
                             gsems[2 + slot])

        def wait_gather(slot):
            pltpu.make_async_copy(a_hbm.at[idx_v.at[0, 0]], bufs_a.at[slot],
                                  gsems[slot]).wait()
            pltpu.make_async_copy(b_hbm.at[idx_v.at[1, 0]], bufs_b.at[slot],
                                  gsems[2 + slot]).wait()

        def store(j, slot):
            base = wid * ew + j * K
            pltpu.async_copy(bufs_a.at[slot], oa_hbm.at[pl.ds(base, K)],
                             wsems[slot])
            pltpu.async_copy(bufs_b.at[slot], ob_hbm.at[pl.ds(base, K)],
                             wsems[2 + slot])

        def wait_store(slot):
            base = wid * ew
            pltpu.make_async_copy(bufs_a.at[slot], oa_hbm.at[pl.ds(base, K)],
                                  wsems[slot]).wait()
            pltpu.make_async_copy(bufs_b.at[slot], ob_hbm.at[pl.ds(base, K)],
                                  wsems[2 + slot]).wait()

        gather(0, 0)
        gather(1, 1)

        @pl.loop(0, (C - 1) // 2)
        def _(m):
            j = 2 * m
            wait_gather(0)
            store(j, 0)

            @pl.when(j + 2 < C)
            def _():
                wait_store(0)
                gather(j + 2, 0)

            wait_gather(1)
            store(j + 1, 1)

            @pl.when(j + 3 < C)
            def _():
                wait_store(1)
                gather(j + 3, 1)

        if C % 2 == 1:
            wait_gather(0)
            store(C - 1, 0)
            wait_store(0)
        else:
            wait_store(0)
        wait_store(1)

    return k(a_tab, b_tab, eidx)


def _tc_mm(xx, ww, bn=1000):
    n, d = xx.shape
    h = ww.shape[1]

    def body(x_ref, w_ref, o_ref):
        o_ref[...] = jnp.dot(x_ref[...], w_ref[...],
                             preferred_element_type=jnp.float32)

    return pl.pallas_call(
        body,
        grid=(n // bn,),
        in_specs=[pl.BlockSpec((bn, d), lambda i: (i, 0)),
                  pl.BlockSpec((d, h), lambda i: (0, 0))],
        out_specs=pl.BlockSpec((bn, h), lambda i: (i, 0)),
        out_shape=jax.ShapeDtypeStruct((n, h), jnp.float32),
    )(xx, ww)


def _tc_scale(h1, cnt, bn=1000):
    n, h = h1.shape

    def body(h_ref, c_ref, o_ref, dinv_ref):
        deg = 1.0 + c_ref[0, :, 0:1] + c_ref[1, :, 0:1]
        dinv = lax.rsqrt(deg)
        o_ref[...] = h_ref[...] * dinv
        dinv_ref[...] = dinv

    return pl.pallas_call(
        body,
        grid=(n // bn,),
        in_specs=[pl.BlockSpec((bn, h), lambda i: (i, 0)),
                  pl.BlockSpec((NC, bn, LANES), lambda i: (0, i, 0))],
        out_specs=[pl.BlockSpec((bn, h), lambda i: (i, 0)),
                   pl.BlockSpec((bn, 1), lambda i: (i, 0))],
        out_shape=[jax.ShapeDtypeStruct((n, h), jnp.float32),
                   jax.ShapeDtypeStruct((n, 1), jnp.float32)],
    )(h1, cnt)


def _tc_layer(agg, hs, dinv, bias, ww, relu, bn=1000):
    n, h = hs.shape

    def body(a_ref, hs_ref, dinv_ref, b_ref, w_ref, o_ref):
        z = dinv_ref[...] * (a_ref[0] + a_ref[1] + hs_ref[...]) + b_ref[...]
        if relu:
            z = jnp.maximum(z, 0.0)
        o_ref[...] = jnp.dot(z, w_ref[...],
                             preferred_element_type=jnp.float32) * dinv_ref[...]

    return pl.pallas_call(
        body,
        grid=(n // bn,),
        in_specs=[pl.BlockSpec((NC, bn, h), lambda i: (0, i, 0)),
                  pl.BlockSpec((bn, h), lambda i: (i, 0)),
                  pl.BlockSpec((bn, 1), lambda i: (i, 0)),
                  pl.BlockSpec((1, h), lambda i: (0, 0)),
                  pl.BlockSpec((h, h), lambda i: (0, 0))],
        out_specs=pl.BlockSpec((bn, h), lambda i: (i, 0)),
        out_shape=jax.ShapeDtypeStruct((n, h), jnp.float32),
    )(agg, hs, dinv, bias, ww)


def _tc_decode_tables(agg, hs, dinv, bias, wa, wb, l1b, bn=1000):
    n, h = hs.shape

    def body(a_ref, hs_ref, dinv_ref, b_ref, wa_ref, wb_ref, l1b_ref,
             oa_ref, ob_ref):
        z = dinv_ref[...] * (a_ref[0] + a_ref[1] + hs_ref[...]) + b_ref[...]
        oa_ref[...] = jnp.dot(z, wa_ref[...],
                              preferred_element_type=jnp.float32) + l1b_ref[...]
        ob_ref[...] = jnp.dot(z, wb_ref[...],
                              preferred_element_type=jnp.float32)

    return pl.pallas_call(
        body,
        grid=(n // bn,),
        in_specs=[pl.BlockSpec((NC, bn, h), lambda i: (0, i, 0)),
                  pl.BlockSpec((bn, h), lambda i: (i, 0)),
                  pl.BlockSpec((bn, 1), lambda i: (i, 0)),
                  pl.BlockSpec((1, h), lambda i: (0, 0)),
                  pl.BlockSpec((h, h), lambda i: (0, 0)),
                  pl.BlockSpec((h, h), lambda i: (0, 0)),
                  pl.BlockSpec((1, h), lambda i: (0, 0))],
        out_specs=[pl.BlockSpec((bn, h), lambda i: (i, 0)),
                   pl.BlockSpec((bn, h), lambda i: (i, 0))],
        out_shape=[jax.ShapeDtypeStruct((n, h), jnp.float32),
                   jax.ShapeDtypeStruct((n, h), jnp.float32)],
    )(agg, hs, dinv, bias, wa, wb, l1b)


def _tc_decode(asrc, bdst, w2, b2, bn=4000):
    e, h = asrc.shape

    def body(a_ref, b_ref, w_ref, bb_ref, o_ref):
        z = jnp.maximum(a_ref[...] + b_ref[...], 0.0)
        o_ref[...] = jax.nn.sigmoid(
            jnp.dot(z, w_ref[...], preferred_element_type=jnp.float32)
            + bb_ref[...])

    return pl.pallas_call(
        body,
        grid=(e // bn,),
        in_specs=[pl.BlockSpec((bn, h), lambda i: (i, 0)),
                  pl.BlockSpec((bn, h), lambda i: (i, 0)),
                  pl.BlockSpec((h, 1), lambda i: (0, 0)),
                  pl.BlockSpec((1, 1), lambda i: (0, 0))],
        out_specs=pl.BlockSpec((bn, 1), lambda i: (i, 0)),
        out_shape=jax.ShapeDtypeStruct((e, 1), jnp.float32),
    )(asrc, bdst, w2, b2)


def kernel(x, edge_index, W1, b1, W2, b2, lin1_W, lin1_b, lin2_W, lin2_b):
    n, _ = x.shape
    h = W1.shape[1]
    e = edge_index.shape[1]

    ew = e // NW
    K = 80
    C = ew // K
    assert ew == C * K and C % 2 == 1

    eidx = jnp.stack([edge_index[0].reshape(NW, C, K),
                      edge_index[1].reshape(NW, C, K)], axis=1)

    cnt = _sc_counts(eidx, n)
    h1 = _tc_mm(x, W1)
    h1s, dinv = _tc_scale(h1, cnt)

    agg1 = _sc_agg(h1s, eidx)
    h2s = _tc_layer(agg1, h1s, dinv, b1.reshape(1, h), W2, relu=True)

    agg2 = _sc_agg(h2s, eidx)
    a_tab, b_tab = _tc_decode_tables(agg2, h2s, dinv, b2.reshape(1, h),
                                     lin1_W[:h], lin1_W[h:],
                                     lin1_b.reshape(1, h))

    asrc, bdst = _sc_gather2(a_tab, b_tab, eidx)
    return _tc_decode(asrc, bdst, lin2_W, lin2_b.reshape(1, 1))


# speedup vs baseline: 28.0204x; 2.2594x over previous
import functools

import jax
import jax.numpy as jnp
from jax import lax
from jax.experimental import pallas as pl
from jax.experimental.pallas import tpu as pltpu
from jax.experimental.pallas import tpu_sc as plsc

NC = 2
NS = 16
LANES = 16
NW = NC * NS


def _vector_mesh():
    return plsc.VectorSubcoreMesh(core_axis_name="c", subcore_axis_name="s")


_SC_PARAMS = pltpu.CompilerParams(use_tc_tiling_on_sc=False)
_SC_PARAMS_NOLAYOUT = pltpu.CompilerParams(use_tc_tiling_on_sc=False,
                                           needs_layout_passes=False)


def _padded_rows(n):
    return ((n + NS * 8 - 1) // (NS * 8)) * (NS * 8)


def _sc_counts(eidx, n):
    _, _, C, K = eidx.shape
    np_ = _padded_rows(n)
    rpt = np_ // NS

    @functools.partial(
        pl.kernel,
        out_type=jax.ShapeDtypeStruct((NC, np_, LANES), jnp.float32),
        mesh=_vector_mesh(),
        compiler_params=_SC_PARAMS,
        scratch_types=[
            pltpu.VMEM((C, K), jnp.int32),
            pltpu.VMEM((K, LANES), jnp.float32),
            pltpu.VMEM((rpt, LANES), jnp.float32),
            pltpu.VMEM_SHARED((np_, LANES), jnp.float32),
        ],
    )
    def k(eidx_hbm, out_hbm, idx_v, ones_v, zero_v, acc_s):
        cid = lax.axis_index("c")
        sid = lax.axis_index("s")
        wid = sid * NC + cid

        pltpu.sync_copy(eidx_hbm.at[1, wid], idx_v)

        @pl.loop(0, K)
        def _(i):
            ones_v[i] = jnp.ones((LANES,), jnp.float32)

        @pl.loop(0, rpt)
        def _(i):
            zero_v[i] = jnp.zeros((LANES,), jnp.float32)

        base = sid * rpt
        pltpu.sync_copy(zero_v, acc_s.at[pl.ds(base, rpt)])
        plsc.subcore_barrier()

        @pl.loop(0, C)
        def _(j):
            pltpu.sync_copy(ones_v, acc_s.at[idx_v.at[j]], add=True)

        plsc.subcore_barrier()
        pltpu.sync_copy(acc_s.at[pl.ds(base, rpt)],
                        out_hbm.at[cid, pl.ds(base, rpt)])

    return k(eidx)


def _sc_agg(table, eidx):
    n, h = table.shape
    _, _, C, K = eidx.shape
    np_ = _padded_rows(n)
    rpt = np_ // NS

    @functools.partial(
        pl.kernel,
        out_type=jax.ShapeDtypeStruct((NC, np_, h), jnp.float32),
        mesh=_vector_mesh(),
        compiler_params=_SC_PARAMS,
        scratch_types=[
            pltpu.VMEM((2, C, K), jnp.int32),
            pltpu.VMEM((K, h), jnp.float32),
            pltpu.VMEM((K, h), jnp.float32),
            pltpu.VMEM((rpt, h), jnp.float32),
            pltpu.VMEM_SHARED((np_, h), jnp.float32),
            pltpu.SemaphoreType.DMA,
            pltpu.SemaphoreType.DMA,
        ],
    )
    def k(table_hbm, eidx_hbm, out_hbm, idx_v, rows_a, rows_b, zero_v,
          acc_s, sem_a, sem_b):
        cid = lax.axis_index("c")
        sid = lax.axis_index("s")
        wid = sid * NC + cid

        cp_s = pltpu.async_copy(eidx_hbm.at[0, wid], idx_v.at[0], sem_a)
        cp_d = pltpu.async_copy(eidx_hbm.at[1, wid], idx_v.at[1], sem_b)

        @pl.loop(0, rpt)
        def _(i):
            @pl.loop(0, h, step=LANES)
            def _(c):
                zero_v[i, pl.ds(c, LANES)] = jnp.zeros((LANES,), jnp.float32)

        base = sid * rpt
        cp_s.wait()
        cp_d.wait()
        pltpu.sync_copy(zero_v, acc_s.at[pl.ds(base, rpt)])
        plsc.subcore_barrier()

        pltpu.async_copy(table_hbm.at[idx_v.at[0, 0]], rows_a, sem_a)
        pltpu.async_copy(table_hbm.at[idx_v.at[0, 1]], rows_b, sem_b)

        @pl.loop(0, (C - 1) // 2)
        def _(m):
            j = 2 * m
            pltpu.make_async_copy(table_hbm.at[idx_v.at[0, 0]],
                                  rows_a, sem_a).wait()
            pltpu.sync_copy(rows_a, acc_s.at[idx_v.at[1, j]], add=True)

            @pl.when(j + 2 < C)
            def _():
                pltpu.async_copy(table_hbm.at[idx_v.at[0, j + 2]],
                                 rows_a, sem_a)

            pltpu.make_async_copy(table_hbm.at[idx_v.at[0, 1]],
                                  rows_b, sem_b).wait()
            pltpu.sync_copy(rows_b, acc_s.at[idx_v.at[1, j + 1]], add=True)

            @pl.when(j + 3 < C)
            def _():
                pltpu.async_copy(table_hbm.at[idx_v.at[0, j + 3]],
                                 rows_b, sem_b)

        if C % 2 == 1:
            pltpu.make_async_copy(table_hbm.at[idx_v.at[0, 0]],
                                  rows_a, sem_a).wait()
            pltpu.sync_copy(rows_a, acc_s.at[idx_v.at[1, C - 1]], add=True)

        plsc.subcore_barrier()
        pltpu.sync_copy(acc_s.at[pl.ds(base, rpt)],
                        out_hbm.at[cid, pl.ds(base, rpt)])

    return k(table, eidx)


def _sc_decode(a_tab, b_tab, wvec, biasvec, eidx):
    n, h = a_tab.shape
    _, _, C, K = eidx.shape
    ew = C * K
    e = NW * ew
    nslice = h // LANES
    ngrp = K // LANES

    @functools.partial(
        pl.kernel,
        out_type=jax.ShapeDtypeStruct((e,), jnp.float32),
        mesh=_vector_mesh(),
        compiler_params=_SC_PARAMS_NOLAYOUT,
        scratch_types=[
            pltpu.VMEM((2, C, K), jnp.int32),
            pltpu.VMEM((2, K, h), jnp.float32),
            pltpu.VMEM((2, K, h), jnp.float32),
            pltpu.VMEM((2, K), jnp.float32),
            pltpu.VMEM((h,), jnp.float32),
            pltpu.VMEM((LANES,), jnp.float32),
            [pltpu.SemaphoreType.DMA] * 4,
        ],
    )
    def k(a_hbm, b_hbm, w_hbm, bias_hbm, eidx_hbm, out_hbm, idx_v,
          bufs_a, bufs_b, out_v, w_v, bias_v, gsems):
        cid = lax.axis_index("c")
        sid = lax.axis_index("s")
        wid = sid * NC + cid

        pltpu.sync_copy(eidx_hbm.at[0, wid], idx_v.at[0])
        pltpu.sync_copy(eidx_hbm.at[1, wid], idx_v.at[1])
        pltpu.sync_copy(w_hbm.at[0], w_v)
        pltpu.sync_copy(bias_hbm, bias_v)

        ws = [w_v[pl.ds(c * LANES, LANES)] for c in range(nslice)]
        bias = bias_v[...]
        lane = lax.iota(jnp.int32, LANES)
        masks = [lane == li for li in range(LANES)]

        def gather(j, slot):
            pltpu.async_copy(a_hbm.at[idx_v.at[0, j]], bufs_a.at[slot],
                             gsems[slot])
            pltpu.async_copy(b_hbm.at[idx_v.at[1, j]], bufs_b.at[slot],
                             gsems[2 + slot])

        def wait_gather(slot):
            pltpu.make_async_copy(a_hbm.at[idx_v.at[0, 0]], bufs_a.at[slot],
                                  gsems[slot]).wait()
            pltpu.make_async_copy(b_hbm.at[idx_v.at[1, 0]], bufs_b.at[slot],
                                  gsems[2 + slot]).wait()

        def compute_store(j, slot):
            a_v = bufs_a.at[slot]
            b_v = bufs_b.at[slot]
            o_v = out_v.at[slot]

            @pl.loop(0, ngrp)
            def _(g):
                logits = bias
                for li in range(LANES):
                    ei = g * LANES + li
                    acc = bias
                    for c in range(nslice):
                        s = (a_v[ei, pl.ds(c * LANES, LANES)]
                             + b_v[ei, pl.ds(c * LANES, LANES)])
                        acc = acc + ws[c] * jnp.maximum(s, 0.0)
                    tot = jnp.broadcast_to(jnp.sum(acc), (LANES,))
                    logits = jnp.where(masks[li], tot, logits)
                o_v[pl.ds(g * LANES, LANES)] = 1.0 / (1.0 + jnp.exp(-logits))

            pltpu.sync_copy(o_v, out_hbm.at[pl.ds(wid * ew + j * K, K)])

        gather(0, 0)
        gather(1, 1)

        @pl.loop(0, (C - 1) // 2)
        def _(m):
            j = 2 * m
            wait_gather(0)
            compute_store(j, 0)

            @pl.when(j + 2 < C)
            def _():
                gather(j + 2, 0)

            wait_gather(1)
            compute_store(j + 1, 1)

            @pl.when(j + 3 < C)
            def _():
                gather(j + 3, 1)

        wait_gather(0)
        compute_store(C - 1, 0)

    return k(a_tab, b_tab, wvec, biasvec, eidx)


def _tc_mm(xx, ww, bn=1000):
    n, d = xx.shape
    h = ww.shape[1]

    def body(x_ref, w_ref, o_ref):
        o_ref[...] = jnp.dot(x_ref[...], w_ref[...],
                             preferred_element_type=jnp.float32)

    return pl.pallas_call(
        body,
        grid=(n // bn,),
        in_specs=[pl.BlockSpec((bn, d), lambda i: (i, 0)),
                  pl.BlockSpec((d, h), lambda i: (0, 0))],
        out_specs=pl.BlockSpec((bn, h), lambda i: (i, 0)),
        out_shape=jax.ShapeDtypeStruct((n, h), jnp.float32),
    )(xx, ww)


def _tc_scale(h1, cnt, bn=1000):
    n, h = h1.shape

    def body(h_ref, c_ref, o_ref, dinv_ref):
        deg = 1.0 + c_ref[0, :, 0:1] + c_ref[1, :, 0:1]
        dinv = lax.rsqrt(deg)
        o_ref[...] = h_ref[...] * dinv
        dinv_ref[...] = dinv

    return pl.pallas_call(
        body,
        grid=(n // bn,),
        in_specs=[pl.BlockSpec((bn, h), lambda i: (i, 0)),
                  pl.BlockSpec((NC, bn, LANES), lambda i: (0, i, 0))],
        out_specs=[pl.BlockSpec((bn, h), lambda i: (i, 0)),
                   pl.BlockSpec((bn, 1), lambda i: (i, 0))],
        out_shape=[jax.ShapeDtypeStruct((n, h), jnp.float32),
                   jax.ShapeDtypeStruct((n, 1), jnp.float32)],
    )(h1, cnt)


def _tc_layer(agg, hs, dinv, bias, ww, relu, bn=1000):
    n, h = hs.shape

    def body(a_ref, hs_ref, dinv_ref, b_ref, w_ref, o_ref):
        z = dinv_ref[...] * (a_ref[0] + a_ref[1] + hs_ref[...]) + b_ref[...]
        if relu:
            z = jnp.maximum(z, 0.0)
        o_ref[...] = jnp.dot(z, w_ref[...],
                             preferred_element_type=jnp.float32) * dinv_ref[...]

    return pl.pallas_call(
        body,
        grid=(n // bn,),
        in_specs=[pl.BlockSpec((NC, bn, h), lambda i: (0, i, 0)),
                  pl.BlockSpec((bn, h), lambda i: (i, 0)),
                  pl.BlockSpec((bn, 1), lambda i: (i, 0)),
                  pl.BlockSpec((1, h), lambda i: (0, 0)),
                  pl.BlockSpec((h, h), lambda i: (0, 0))],
        out_specs=pl.BlockSpec((bn, h), lambda i: (i, 0)),
        out_shape=jax.ShapeDtypeStruct((n, h), jnp.float32),
    )(agg, hs, dinv, bias, ww)


def _tc_decode_tables(agg, hs, dinv, bias, wa, wb, l1b, bn=1000):
    n, h = hs.shape

    def body(a_ref, hs_ref, dinv_ref, b_ref, wa_ref, wb_ref, l1b_ref,
             oa_ref, ob_ref):
        z = dinv_ref[...] * (a_ref[0] + a_ref[1] + hs_ref[...]) + b_ref[...]
        oa_ref[...] = jnp.dot(z, wa_ref[...],
                              preferred_element_type=jnp.float32) + l1b_ref[...]
        ob_ref[...] = jnp.dot(z, wb_ref[...],
                              preferred_element_type=jnp.float32)

    return pl.pallas_call(
        body,
        grid=(n // bn,),
        in_specs=[pl.BlockSpec((NC, bn, h), lambda i: (0, i, 0)),
                  pl.BlockSpec((bn, h), lambda i: (i, 0)),
                  pl.BlockSpec((bn, 1), lambda i: (i, 0)),
                  pl.BlockSpec((1, h), lambda i: (0, 0)),
                  pl.BlockSpec((h, h), lambda i: (0, 0)),
                  pl.BlockSpec((h, h), lambda i: (0, 0)),
                  pl.BlockSpec((1, h), lambda i: (0, 0))],
        out_specs=[pl.BlockSpec((bn, h), lambda i: (i, 0)),
                   pl.BlockSpec((bn, h), lambda i: (i, 0))],
        out_shape=[jax.ShapeDtypeStruct((n, h), jnp.float32),
                   jax.ShapeDtypeStruct((n, h), jnp.float32)],
    )(agg, hs, dinv, bias, wa, wb, l1b)


def _tc_decode(asrc, bdst, w2, b2, bn=4000):
    e, h = asrc.shape

    def body(a_ref, b_ref, w_ref, bb_ref, o_ref):
        z = jnp.maximum(a_ref[...] + b_ref[...], 0.0)
        o_ref[...] = jax.nn.sigmoid(
            jnp.dot(z, w_ref[...], preferred_element_type=jnp.float32)
            + bb_ref[...])

    return pl.pallas_call(
        body,
        grid=(e // bn,),
        in_specs=[pl.BlockSpec((bn, h), lambda i: (i, 0)),
                  pl.BlockSpec((bn, h), lambda i: (i, 0)),
                  pl.BlockSpec((h, 1), lambda i: (0, 0)),
                  pl.BlockSpec((1, 1), lambda i: (0, 0))],
        out_specs=pl.BlockSpec((bn, 1), lambda i: (i, 0)),
        out_shape=jax.ShapeDtypeStruct((e, 1), jnp.float32),
    )(asrc, bdst, w2, b2)


def kernel(x, edge_index, W1, b1, W2, b2, lin1_W, lin1_b, lin2_W, lin2_b):
    n, _ = x.shape
    h = W1.shape[1]
    e = edge_index.shape[1]

    ew = e // NW
    K = 80
    C = ew // K
    assert ew == C * K and C % 2 == 1

    eidx = edge_index.reshape(2, NW, C, K)

    cnt = _sc_counts(eidx, n)
    h1 = _tc_mm(x, W1)
    h1s, dinv = _tc_scale(h1, cnt)

    agg1 = _sc_agg(h1s, eidx)
    h2s = _tc_layer(agg1, h1s, dinv, b1.reshape(1, h), W2, relu=True)

    agg2 = _sc_agg(h2s, eidx)
    a_tab, b_tab = _tc_decode_tables(agg2, h2s, dinv, b2.reshape(1, h),
                                     lin1_W[:h], lin1_W[h:],
                                     lin1_b.reshape(1, h))

    wvec = lin2_W.reshape(1, h)
    biasvec = jnp.full((LANES,), lin2_b[0] / LANES, dtype=jnp.float32)
    out = _sc_decode(a_tab, b_tab, wvec, biasvec, eidx)
    return out.reshape(e, 1)


# speedup vs baseline: 32.6332x; 1.1646x over previous
import functools

import jax
import jax.numpy as jnp
from jax import lax
from jax.experimental import pallas as pl
from jax.experimental.pallas import tpu as pltpu
from jax.experimental.pallas import tpu_sc as plsc

NC = 2
NS = 16
LANES = 16
NW = NC * NS


def _vector_mesh():
    return plsc.VectorSubcoreMesh(core_axis_name="c", subcore_axis_name="s")


_SC_PARAMS = pltpu.CompilerParams(use_tc_tiling_on_sc=False)
_SC_PARAMS_NOLAYOUT = pltpu.CompilerParams(use_tc_tiling_on_sc=False,
                                           needs_layout_passes=False)


def _padded_rows(n):
    return ((n + NS * 8 - 1) // (NS * 8)) * (NS * 8)


def _sc_counts(eidx, n):
    _, _, C, K = eidx.shape
    np_ = _padded_rows(n)
    rpt = np_ // NS

    @functools.partial(
        pl.kernel,
        out_type=jax.ShapeDtypeStruct((NC, np_, LANES), jnp.float32),
        mesh=_vector_mesh(),
        compiler_params=_SC_PARAMS,
        scratch_types=[
            pltpu.VMEM((C, K), jnp.int32),
            pltpu.VMEM((K, LANES), jnp.float32),
            pltpu.VMEM((rpt, LANES), jnp.float32),
            pltpu.VMEM_SHARED((np_, LANES), jnp.float32),
        ],
    )
    def k(eidx_hbm, out_hbm, idx_v, ones_v, zero_v, acc_s):
        cid = lax.axis_index("c")
        sid = lax.axis_index("s")
        wid = sid * NC + cid

        pltpu.sync_copy(eidx_hbm.at[1, wid], idx_v)

        @pl.loop(0, K)
        def _(i):
            ones_v[i] = jnp.ones((LANES,), jnp.float32)

        @pl.loop(0, rpt)
        def _(i):
            zero_v[i] = jnp.zeros((LANES,), jnp.float32)

        base = sid * rpt
        pltpu.sync_copy(zero_v, acc_s.at[pl.ds(base, rpt)])
        plsc.subcore_barrier()

        @pl.loop(0, C)
        def _(j):
            pltpu.sync_copy(ones_v, acc_s.at[idx_v.at[j]], add=True)

        plsc.subcore_barrier()
        pltpu.sync_copy(acc_s.at[pl.ds(base, rpt)],
                        out_hbm.at[cid, pl.ds(base, rpt)])

    return k(eidx)


def _sc_agg(table, eidx):
    n, h = table.shape
    _, _, C, K = eidx.shape
    np_ = _padded_rows(n)
    rpt = np_ // NS

    nbuf = 4
    assert (C - 1) % nbuf == 0

    @functools.partial(
        pl.kernel,
        out_type=jax.ShapeDtypeStruct((NC, np_, h), jnp.float32),
        mesh=_vector_mesh(),
        compiler_params=_SC_PARAMS,
        scratch_types=[
            pltpu.VMEM((2, C, K), jnp.int32),
            pltpu.VMEM((nbuf, K, h), jnp.float32),
            pltpu.VMEM((rpt, h), jnp.float32),
            pltpu.VMEM_SHARED((np_, h), jnp.float32),
            [pltpu.SemaphoreType.DMA] * nbuf,
            [pltpu.SemaphoreType.DMA] * nbuf,
            pltpu.SemaphoreType.DMA,
        ],
    )
    def k(table_hbm, eidx_hbm, out_hbm, idx_v, bufs, zero_v,
          acc_s, gsems, ssems, isem):
        cid = lax.axis_index("c")
        sid = lax.axis_index("s")
        wid = sid * NC + cid

        cp_s = pltpu.async_copy(eidx_hbm.at[0, wid], idx_v.at[0], isem)
        cp_d = pltpu.async_copy(eidx_hbm.at[1, wid], idx_v.at[1], isem)

        @pl.loop(0, rpt)
        def _(i):
            @pl.loop(0, h, step=LANES)
            def _(c):
                zero_v[i, pl.ds(c, LANES)] = jnp.zeros((LANES,), jnp.float32)

        base = sid * rpt
        cp_s.wait()
        cp_d.wait()
        pltpu.sync_copy(zero_v, acc_s.at[pl.ds(base, rpt)])
        plsc.subcore_barrier()

        def gather(j, b):
            pltpu.async_copy(table_hbm.at[idx_v.at[0, j]], bufs.at[b],
                             gsems[b])

        def wait_gather(b):
            pltpu.make_async_copy(table_hbm.at[idx_v.at[0, 0]], bufs.at[b],
                                  gsems[b]).wait()

        def scatter(j, b):
            pltpu.async_copy(bufs.at[b], acc_s.at[idx_v.at[1, j]], ssems[b],
                             add=True)

        def wait_scatter(b):
            pltpu.make_async_copy(bufs.at[b], acc_s.at[idx_v.at[1, 0]],
                                  ssems[b]).wait()

        for b in range(nbuf):
            gather(b, b)

        @pl.loop(0, (C - 1) // nbuf)
        def _(m):
            j0 = nbuf * m
            for b in range(nbuf):
                j = j0 + b
                wait_gather(b)
                scatter(j, b)

                @pl.when(j + nbuf < C)
                def _():
                    wait_scatter(b)
                    gather(j + nbuf, b)

        wait_gather(0)
        scatter(C - 1, 0)
        for b in range(nbuf):
            wait_scatter(b)

        plsc.subcore_barrier()
        pltpu.sync_copy(acc_s.at[pl.ds(base, rpt)],
                        out_hbm.at[cid, pl.ds(base, rpt)])

    return k(table, eidx)


def _sc_decode(a_tab, b_tab, wvec, biasvec, eidx):
    n, h = a_tab.shape
    _, _, C, K = eidx.shape
    ew = C * K
    e = NW * ew
    nslice = h // LANES
    ngrp = K // LANES

    @functools.partial(
        pl.kernel,
        out_type=jax.ShapeDtypeStruct((e,), jnp.float32),
        mesh=_vector_mesh(),
        compiler_params=_SC_PARAMS_NOLAYOUT,
        scratch_types=[
            pltpu.VMEM((2, C, K), jnp.int32),
            pltpu.VMEM((2, K, h), jnp.float32),
            pltpu.VMEM((2, K, h), jnp.float32),
            pltpu.VMEM((2, K), jnp.float32),
            pltpu.VMEM((h,), jnp.float32),
            pltpu.VMEM((LANES,), jnp.float32),
            [pltpu.SemaphoreType.DMA] * 4,
        ],
    )
    def k(a_hbm, b_hbm, w_hbm, bias_hbm, eidx_hbm, out_hbm, idx_v,
          bufs_a, bufs_b, out_v, w_v, bias_v, gsems):
        cid = lax.axis_index("c")
        sid = lax.axis_index("s")
        wid = sid * NC + cid

        pltpu.sync_copy(eidx_hbm.at[0, wid], idx_v.at[0])
        pltpu.sync_copy(eidx_hbm.at[1, wid], idx_v.at[1])
        pltpu.sync_copy(w_hbm.at[0], w_v)
        pltpu.sync_copy(bias_hbm, bias_v)

        ws = [w_v[pl.ds(c * LANES, LANES)] for c in range(nslice)]
        bias = bias_v[...]
        lane = lax.iota(jnp.int32, LANES)
        masks = [lane == li for li in range(LANES)]

        def gather(j, slot):
            pltpu.async_copy(a_hbm.at[idx_v.at[0, j]], bufs_a.at[slot],
                             gsems[slot])
            pltpu.async_copy(b_hbm.at[idx_v.at[1, j]], bufs_b.at[slot],
                             gsems[2 + slot])

        def wait_gather(slot):
            pltpu.make_async_copy(a_hbm.at[idx_v.at[0, 0]], bufs_a.at[slot],
                                  gsems[slot]).wait()
            pltpu.make_async_copy(b_hbm.at[idx_v.at[1, 0]], bufs_b.at[slot],
                                  gsems[2 + slot]).wait()

        def compute_store(j, slot):
            a_v = bufs_a.at[slot]
            b_v = bufs_b.at[slot]
            o_v = out_v.at[slot]

            @pl.loop(0, ngrp)
            def _(g):
                logits = bias
                for li in range(LANES):
                    ei = g * LANES + li
                    acc = bias
                    for c in range(nslice):
                        s = (a_v[ei, pl.ds(c * LANES, LANES)]
                             + b_v[ei, pl.ds(c * LANES, LANES)])
                        acc = acc + ws[c] * jnp.maximum(s, 0.0)
                    tot = jnp.broadcast_to(jnp.sum(acc), (LANES,))
                    logits = jnp.where(masks[li], tot, logits)
                o_v[pl.ds(g * LANES, LANES)] = 1.0 / (1.0 + jnp.exp(-logits))

            pltpu.sync_copy(o_v, out_hbm.at[pl.ds(wid * ew + j * K, K)])

        gather(0, 0)
        gather(1, 1)

        @pl.loop(0, (C - 1) // 2)
        def _(m):
            j = 2 * m
            wait_gather(0)
            compute_store(j, 0)

            @pl.when(j + 2 < C)
            def _():
                gather(j + 2, 0)

            wait_gather(1)
            compute_store(j + 1, 1)

            @pl.when(j + 3 < C)
            def _():
                gather(j + 3, 1)

        wait_gather(0)
        compute_store(C - 1, 0)

    return k(a_tab, b_tab, wvec, biasvec, eidx)


def _tc_mm(xx, ww, bn=1000):
    n, d = xx.shape
    h = ww.shape[1]

    def body(x_ref, w_ref, o_ref):
        o_ref[...] = jnp.dot(x_ref[...], w_ref[...],
                             preferred_element_type=jnp.float32)

    return pl.pallas_call(
        body,
        grid=(n // bn,),
        in_specs=[pl.BlockSpec((bn, d), lambda i: (i, 0)),
                  pl.BlockSpec((d, h), lambda i: (0, 0))],
        out_specs=pl.BlockSpec((bn, h), lambda i: (i, 0)),
        out_shape=jax.ShapeDtypeStruct((n, h), jnp.float32),
    )(xx, ww)


def _tc_scale(h1, cnt, bn=1000):
    n, h = h1.shape

    def body(h_ref, c_ref, o_ref, dinv_ref):
        deg = 1.0 + c_ref[0, :, 0:1] + c_ref[1, :, 0:1]
        dinv = lax.rsqrt(deg)
        o_ref[...] = h_ref[...] * dinv
        dinv_ref[...] = dinv

    return pl.pallas_call(
        body,
        grid=(n // bn,),
        in_specs=[pl.BlockSpec((bn, h), lambda i: (i, 0)),
                  pl.BlockSpec((NC, bn, LANES), lambda i: (0, i, 0))],
        out_specs=[pl.BlockSpec((bn, h), lambda i: (i, 0)),
                   pl.BlockSpec((bn, 1), lambda i: (i, 0))],
        out_shape=[jax.ShapeDtypeStruct((n, h), jnp.float32),
                   jax.ShapeDtypeStruct((n, 1), jnp.float32)],
    )(h1, cnt)


def _tc_layer(agg, hs, dinv, bias, ww, relu, bn=1000):
    n, h = hs.shape

    def body(a_ref, hs_ref, dinv_ref, b_ref, w_ref, o_ref):
        z = dinv_ref[...] * (a_ref[0] + a_ref[1] + hs_ref[...]) + b_ref[...]
        if relu:
            z = jnp.maximum(z, 0.0)
        o_ref[...] = jnp.dot(z, w_ref[...],
                             preferred_element_type=jnp.float32) * dinv_ref[...]

    return pl.pallas_call(
        body,
        grid=(n // bn,),
        in_specs=[pl.BlockSpec((NC, bn, h), lambda i: (0, i, 0)),
                  pl.BlockSpec((bn, h), lambda i: (i, 0)),
                  pl.BlockSpec((bn, 1), lambda i: (i, 0)),
                  pl.BlockSpec((1, h), lambda i: (0, 0)),
                  pl.BlockSpec((h, h), lambda i: (0, 0))],
        out_specs=pl.BlockSpec((bn, h), lambda i: (i, 0)),
        out_shape=jax.ShapeDtypeStruct((n, h), jnp.float32),
    )(agg, hs, dinv, bias, ww)


def _tc_decode_tables(agg, hs, dinv, bias, wa, wb, l1b, bn=1000):
    n, h = hs.shape

    def body(a_ref, hs_ref, dinv_ref, b_ref, wa_ref, wb_ref, l1b_ref,
             oa_ref, ob_ref):
        z = dinv_ref[...] * (a_ref[0] + a_ref[1] + hs_ref[...]) + b_ref[...]
        oa_ref[...] = jnp.dot(z, wa_ref[...],
                              preferred_element_type=jnp.float32) + l1b_ref[...]
        ob_ref[...] = jnp.dot(z, wb_ref[...],
                              preferred_element_type=jnp.float32)

    return pl.pallas_call(
        body,
        grid=(n // bn,),
        in_specs=[pl.BlockSpec((NC, bn, h), lambda i: (0, i, 0)),
                  pl.BlockSpec((bn, h), lambda i: (i, 0)),
                  pl.BlockSpec((bn, 1), lambda i: (i, 0)),
                  pl.BlockSpec((1, h), lambda i: (0, 0)),
                  pl.BlockSpec((h, h), lambda i: (0, 0)),
                  pl.BlockSpec((h, h), lambda i: (0, 0)),
                  pl.BlockSpec((1, h), lambda i: (0, 0))],
        out_specs=[pl.BlockSpec((bn, h), lambda i: (i, 0)),
                   pl.BlockSpec((bn, h), lambda i: (i, 0))],
        out_shape=[jax.ShapeDtypeStruct((n, h), jnp.float32),
                   jax.ShapeDtypeStruct((n, h), jnp.float32)],
    )(agg, hs, dinv, bias, wa, wb, l1b)


def _tc_decode(asrc, bdst, w2, b2, bn=4000):
    e, h = asrc.shape

    def body(a_ref, b_ref, w_ref, bb_ref, o_ref):
        z = jnp.maximum(a_ref[...] + b_ref[...], 0.0)
        o_ref[...] = jax.nn.sigmoid(
            jnp.dot(z, w_ref[...], preferred_element_type=jnp.float32)
            + bb_ref[...])

    return pl.pallas_call(
        body,
        grid=(e // bn,),
        in_specs=[pl.BlockSpec((bn, h), lambda i: (i, 0)),
                  pl.BlockSpec((bn, h), lambda i: (i, 0)),
                  pl.BlockSpec((h, 1), lambda i: (0, 0)),
                  pl.BlockSpec((1, 1), lambda i: (0, 0))],
        out_specs=pl.BlockSpec((bn, 1), lambda i: (i, 0)),
        out_shape=jax.ShapeDtypeStruct((e, 1), jnp.float32),
    )(asrc, bdst, w2, b2)


def kernel(x, edge_index, W1, b1, W2, b2, lin1_W, lin1_b, lin2_W, lin2_b):
    n, _ = x.shape
    h = W1.shape[1]
    e = edge_index.shape[1]

    ew = e // NW
    K = 80
    C = ew // K
    assert ew == C * K and C % 2 == 1

    eidx = edge_index.reshape(2, NW, C, K)

    cnt = _sc_counts(eidx, n)
    h1 = _tc_mm(x, W1)
    h1s, dinv = _tc_scale(h1, cnt)

    agg1 = _sc_agg(h1s, eidx)
    h2s = _tc_layer(agg1, h1s, dinv, b1.reshape(1, h), W2, relu=True)

    agg2 = _sc_agg(h2s, eidx)
    a_tab, b_tab = _tc_decode_tables(agg2, h2s, dinv, b2.reshape(1, h),
                                     lin1_W[:h], lin1_W[h:],
                                     lin1_b.reshape(1, h))

    wvec = lin2_W.reshape(1, h)
    biasvec = jnp.full((LANES,), lin2_b[0] / LANES, dtype=jnp.float32)
    out = _sc_decode(a_tab, b_tab, wvec, biasvec, eidx)
    return out.reshape(e, 1)
